# baseline (speedup 1.0000x reference)
import functools

import jax
import jax.numpy as jnp
from jax import lax
from jax.experimental import pallas as pl
from jax.experimental.pallas import tpu as pltpu
from jax.experimental.pallas import tpu_sc as plsc

_SUB = 256
_LANES = 16


def _router_body(x_ref, w_ref, idx_ref, gate_ref, cnts_ref, laux_ref,
                 cnt_ref, me_ref):
    i = pl.program_id(0)
    nblocks = pl.num_programs(0)
    B = x_ref.shape[0]
    E = w_ref.shape[0]

    @pl.when(i == 0)
    def _init():
        cnt_ref[...] = jnp.zeros_like(cnt_ref)
        me_ref[...] = jnp.zeros_like(me_ref)

    x = x_ref[...]
    w = w_ref[...]
    logits = jax.lax.dot_general(
        x, w, (((1,), (1,)), ((), ())),
        preferred_element_type=jnp.float32)

    rowmax = jnp.max(logits, axis=1, keepdims=True)
    e = jnp.exp(logits - rowmax)
    s = jnp.sum(e, axis=1, keepdims=True)
    gate_ref[...] = 1.0 / s

    lane = jax.lax.broadcasted_iota(jnp.int32, (B, E), 1)
    idx = jnp.min(jnp.where(logits == rowmax, lane, E), axis=1,
                  keepdims=True)
    idx_ref[...] = idx

    mask = (lane == idx).astype(jnp.float32)

    nsub = B // _SUB
    subs = [jnp.sum(mask[k * _SUB:(k + 1) * _SUB], axis=0, keepdims=True)
            for k in range(nsub)]
    cnts_ref[...] = (jnp.concatenate(subs, axis=0)
                     .astype(jnp.int32).reshape(1, nsub, E))

    cnt_ref[...] = cnt_ref[...] + jnp.sum(mask, axis=0, keepdims=True)
    me_ref[...] = me_ref[...] + jnp.sum(e / s, axis=0, keepdims=True)

    @pl.when(i == nblocks - 1)
    def _fin():
        n_tok = B * nblocks
        ce = cnt_ref[...]
        me = me_ref[...]
        laux_ref[...] = (jnp.sum(me * ce, keepdims=True)
                         * (E / (n_tok * n_tok))).reshape(1, 1)


def _make_locations_kernel(N, E, NC, NS):
    NW = NC * NS
    chunk = N // NW
    ngroups = chunk // _LANES
    mesh = plsc.VectorSubcoreMesh(core_axis_name="c", subcore_axis_name="s",
                                  num_cores=NC, num_subcores=NS)

    @functools.partial(
        pl.kernel, mesh=mesh,
        out_type=jax.ShapeDtypeStruct((N,), jnp.int32),
        compiler_params=pltpu.CompilerParams(needs_layout_passes=False),
        scratch_types=[
            pltpu.VMEM((chunk,), jnp.int32),
            pltpu.VMEM((NW, E), jnp.int32),
            pltpu.VMEM((E,), jnp.int32),
            pltpu.VMEM((chunk,), jnp.int32),
        ],
    )
    def _loc_kernel(idx_hbm, cnts_hbm, loc_hbm, idx_v, cnts_v, cnt_v, loc_v):
        w = lax.axis_index("s") * NC + lax.axis_index("c")
        base = w * chunk
        pltpu.sync_copy(idx_hbm.at[pl.ds(base, chunk)], idx_v)
        pltpu.sync_copy(cnts_hbm, cnts_v)

        for c in range(E // _LANES):
            acc = jnp.zeros((_LANES,), jnp.int32)
            for wp in range(NW):
                row = cnts_v[wp, pl.ds(c * _LANES, _LANES)]
                sel = (jnp.full((_LANES,), wp, jnp.int32)
                       < jnp.full((_LANES,), 1, jnp.int32) * w)
                acc = acc + jnp.where(sel, row, 0)
            cnt_v[pl.ds(c * _LANES, _LANES)] = acc

        lane = lax.iota(jnp.int32, _LANES)
        for g in range(ngroups):
            ids = idx_v[pl.ds(g * _LANES, _LANES)]
            rank = jnp.zeros((_LANES,), jnp.int32)
            tot = jnp.ones((_LANES,), jnp.int32)
            for sh in range(1, _LANES):
                back = ids.at[jnp.maximum(lane - sh, 0)].get(
                    mode="promise_in_bounds")
                fwd = ids.at[jnp.minimum(lane + sh, _LANES - 1)].get(
                    mode="promise_in_bounds")
                rank = rank + jnp.where((lane >= sh) & (ids == back), 1, 0)
                tot = (tot
                       + jnp.where((lane >= sh) & (ids == back), 1, 0)
                       + jnp.where((lane + sh < _LANES) & (ids == fwd), 1, 0))
            old = plsc.load_gather(cnt_v, [ids])
            loc_v[pl.ds(g * _LANES, _LANES)] = old + rank
            plsc.store_scatter(cnt_v, [ids], old + tot)
        pltpu.sync_copy(loc_v, loc_hbm.at[pl.ds(base, chunk)])

    return _loc_kernel


def kernel(input, W):
    N, D = input.shape
    E = W.shape[0]
    B = 1024
    nblocks = N // B
    nsub = B // _SUB
    capacity = int(1.0 * ((N + E - 1) // E))

    idx, gate, cnts, laux = pl.pallas_call(
        _router_body,
        grid=(nblocks,),
        in_specs=[
            pl.BlockSpec((B, D), lambda i: (i, 0)),
            pl.BlockSpec((E, D), lambda i: (0, 0)),
        ],
        out_specs=[
            pl.BlockSpec((B, 1), lambda i: (i, 0)),
            pl.BlockSpec((B, 1), lambda i: (i, 0)),
            pl.BlockSpec((1, nsub, E), lambda i: (i, 0, 0)),
            pl.BlockSpec((1, 1), lambda i: (0, 0)),
        ],
        out_shape=[
            jax.ShapeDtypeStruct((N, 1), jnp.int32),
            jax.ShapeDtypeStruct((N, 1), jnp.float32),
            jax.ShapeDtypeStruct((nblocks, nsub, E), jnp.int32),
            jax.ShapeDtypeStruct((1, 1), jnp.float32),
        ],
        scratch_shapes=[
            pltpu.VMEM((1, E), jnp.float32),
            pltpu.VMEM((1, E), jnp.float32),
        ],
    )(input, W)

    loc_kernel = _make_locations_kernel(N, E, 2, 16)
    loc = loc_kernel(idx[:, 0], cnts.reshape(nblocks * nsub, E))

    return (laux[0, 0],
            jnp.asarray(capacity, dtype=jnp.int32),
            jnp.asarray(E, dtype=jnp.int32),
            idx[:, 0],
            loc,
            gate[:, 0])


# speedup vs baseline: 1.0458x; 1.0458x over previous
import functools

import jax
import jax.numpy as jnp
from jax import lax
from jax.experimental import pallas as pl
from jax.experimental.pallas import tpu as pltpu
from jax.experimental.pallas import tpu_sc as plsc

_SUB = 256
_LANES = 16


def _router_body(x_ref, w_ref, idx_ref, gate_ref, pref_ref, laux_ref,
                 cnt_ref, me_ref):
    i = pl.program_id(0)
    nblocks = pl.num_programs(0)
    B = x_ref.shape[0]
    E = w_ref.shape[0]

    @pl.when(i == 0)
    def _init():
        cnt_ref[...] = jnp.zeros_like(cnt_ref)
        me_ref[...] = jnp.zeros_like(me_ref)

    x = x_ref[...]
    w = w_ref[...]
    logits = jax.lax.dot_general(
        x, w, (((1,), (1,)), ((), ())),
        preferred_element_type=jnp.float32)

    rowmax = jnp.max(logits, axis=1, keepdims=True)
    e = jnp.exp(logits - rowmax)
    s = jnp.sum(e, axis=1, keepdims=True)
    gate_ref[...] = 1.0 / s

    lane = jax.lax.broadcasted_iota(jnp.int32, (B, E), 1)
    idx = jnp.min(jnp.where(logits == rowmax, lane, E), axis=1,
                  keepdims=True)
    idx_ref[...] = idx

    mask = (lane == idx).astype(jnp.float32)

    nsub = B // _SUB
    carry = cnt_ref[...]
    rows = []
    for k in range(nsub):
        rows.append(carry)
        carry = carry + jnp.sum(mask[k * _SUB:(k + 1) * _SUB], axis=0,
                                keepdims=True)
    pref_ref[...] = (jnp.concatenate(rows, axis=0)
                     .astype(jnp.int32).reshape(1, nsub, E))
    cnt_ref[...] = carry

    me_ref[...] = me_ref[...] + jnp.sum(e / s, axis=0, keepdims=True)

    @pl.when(i == nblocks - 1)
    def _fin():
        n_tok = B * nblocks
        ce = carry
        me = me_ref[...]
        laux_ref[...] = (jnp.sum(me * ce, keepdims=True)
                         * (E / (n_tok * n_tok))).reshape(1, 1)


def _make_locations_kernel(N, E, NC, NS):
    NW = NC * NS
    chunk = N // NW
    ngroups = chunk // _LANES
    mesh = plsc.VectorSubcoreMesh(core_axis_name="c", subcore_axis_name="s",
                                  num_cores=NC, num_subcores=NS)

    @functools.partial(
        pl.kernel, mesh=mesh,
        out_type=jax.ShapeDtypeStruct((N,), jnp.int32),
        compiler_params=pltpu.CompilerParams(needs_layout_passes=False),
        scratch_types=[
            pltpu.VMEM((chunk,), jnp.int32),
            pltpu.VMEM((E,), jnp.int32),
            pltpu.VMEM((chunk,), jnp.int32),
        ],
    )
    def _loc_kernel(idx_hbm, pref_hbm, loc_hbm, idx_v, cnt_v, loc_v):
        w = lax.axis_index("s") * NC + lax.axis_index("c")
        base = w * chunk
        pltpu.sync_copy(idx_hbm.at[pl.ds(base, chunk)], idx_v)
        pltpu.sync_copy(pref_hbm.at[w], cnt_v)

        for g in range(ngroups):
            ids = idx_v[pl.ds(g * _LANES, _LANES)]
            cnt16, last = plsc.scan_count(ids)
            old = plsc.load_gather(cnt_v, [ids])
            loc_v[pl.ds(g * _LANES, _LANES)] = old + cnt16 - 1
            plsc.store_scatter(cnt_v, [ids], old + cnt16, mask=last)
        pltpu.sync_copy(loc_v, loc_hbm.at[pl.ds(base, chunk)])

    return _loc_kernel


def kernel(input, W):
    N, D = input.shape
    E = W.shape[0]
    B = 1024
    nblocks = N // B
    nsub = B // _SUB
    capacity = int(1.0 * ((N + E - 1) // E))

    idx, gate, pref, laux = pl.pallas_call(
        _router_body,
        grid=(nblocks,),
        in_specs=[
            pl.BlockSpec((B, D), lambda i: (i, 0)),
            pl.BlockSpec((E, D), lambda i: (0, 0)),
        ],
        out_specs=[
            pl.BlockSpec((B, 1), lambda i: (i, 0)),
            pl.BlockSpec((B, 1), lambda i: (i, 0)),
            pl.BlockSpec((1, nsub, E), lambda i: (i, 0, 0)),
            pl.BlockSpec((1, 1), lambda i: (0, 0)),
        ],
        out_shape=[
            jax.ShapeDtypeStruct((N, 1), jnp.int32),
            jax.ShapeDtypeStruct((N, 1), jnp.float32),
            jax.ShapeDtypeStruct((nblocks, nsub, E), jnp.int32),
            jax.ShapeDtypeStruct((1, 1), jnp.float32),
        ],
        scratch_shapes=[
            pltpu.VMEM((1, E), jnp.float32),
            pltpu.VMEM((1, E), jnp.float32),
        ],
    )(input, W)

    loc_kernel = _make_locations_kernel(N, E, 2, 16)
    loc = loc_kernel(idx[:, 0], pref.reshape(nblocks * nsub, E))

    return (laux[0, 0],
            jnp.asarray(capacity, dtype=jnp.int32),
            jnp.asarray(E, dtype=jnp.int32),
            idx[:, 0],
            loc,
            gate[:, 0])


# speedup vs baseline: 1.0511x; 1.0050x over previous
import functools

import jax
import jax.numpy as jnp
from jax import lax
from jax.experimental import pallas as pl
from jax.experimental.pallas import tpu as pltpu
from jax.experimental.pallas import tpu_sc as plsc

_SUB = 256
_LANES = 16


def _router_body(x_ref, w_ref, idx_ref, gate_ref, pref_ref, laux_ref,
                 cnt_ref, me_ref):
    i = pl.program_id(0)
    nblocks = pl.num_programs(0)
    B = x_ref.shape[0]
    E = w_ref.shape[0]

    @pl.when(i == 0)
    def _init():
        cnt_ref[...] = jnp.zeros_like(cnt_ref)
        me_ref[...] = jnp.zeros_like(me_ref)

    x = x_ref[...]
    w = w_ref[...]
    logits = jax.lax.dot_general(
        x, w, (((1,), (1,)), ((), ())),
        preferred_element_type=jnp.float32)

    rowmax = jnp.max(logits, axis=1, keepdims=True)
    e = jnp.exp(logits - rowmax)
    s = jnp.sum(e, axis=1, keepdims=True)
    gate_ref[...] = 1.0 / s

    lane = jax.lax.broadcasted_iota(jnp.int32, (B, E), 1)
    idx = jnp.min(jnp.where(logits == rowmax, lane, E), axis=1,
                  keepdims=True)
    idx_ref[...] = idx

    mask = (lane == idx).astype(jnp.float32)

    nsub = B // _SUB
    carry = cnt_ref[...]
    rows = []
    for k in range(nsub):
        rows.append(carry)
        carry = carry + jnp.sum(mask[k * _SUB:(k + 1) * _SUB], axis=0,
                                keepdims=True)
    pref_ref[...] = (jnp.concatenate(rows, axis=0)
                     .astype(jnp.int32).reshape(1, nsub, E))
    cnt_ref[...] = carry

    me_ref[...] = me_ref[...] + jnp.sum(e / s, axis=0, keepdims=True)

    @pl.when(i == nblocks - 1)
    def _fin():
        n_tok = B * nblocks
        ce = carry
        me = me_ref[...]
        laux_ref[...] = (jnp.sum(me * ce, keepdims=True)
                         * (E / (n_tok * n_tok))).reshape(1, 1)


def _make_locations_kernel(N, E, NC, NS):
    NW = NC * NS
    chunk = N // NW
    ngroups = chunk // _LANES
    mesh = plsc.VectorSubcoreMesh(core_axis_name="c", subcore_axis_name="s",
                                  num_cores=NC, num_subcores=NS)

    @functools.partial(
        pl.kernel, mesh=mesh,
        out_type=jax.ShapeDtypeStruct((N,), jnp.int32),
        compiler_params=pltpu.CompilerParams(needs_layout_passes=False),
        scratch_types=[
            pltpu.VMEM((chunk,), jnp.int32),
            pltpu.VMEM((E,), jnp.int32),
            pltpu.VMEM((chunk,), jnp.int32),
            pltpu.SemaphoreType.DMA,
            pltpu.SemaphoreType.DMA,
        ],
    )
    def _loc_kernel(idx_hbm, pref_hbm, loc_hbm, idx_v, cnt_v, loc_v,
                    sem1, sem2):
        w = lax.axis_index("s") * NC + lax.axis_index("c")
        base = w * chunk
        c1 = pltpu.async_copy(idx_hbm.at[pl.ds(base, chunk)], idx_v, sem1)
        c2 = pltpu.async_copy(pref_hbm.at[w], cnt_v, sem2)
        c1.wait()
        c2.wait()

        for g in range(ngroups):
            ids = idx_v[pl.ds(g * _LANES, _LANES)]
            cnt16, last = plsc.scan_count(ids)
            old = plsc.load_gather(cnt_v, [ids])
            loc_v[pl.ds(g * _LANES, _LANES)] = old + cnt16 - 1
            plsc.store_scatter(cnt_v, [ids], old + cnt16, mask=last)
        pltpu.sync_copy(loc_v, loc_hbm.at[pl.ds(base, chunk)])

    return _loc_kernel


def kernel(input, W):
    N, D = input.shape
    E = W.shape[0]
    B = 1024
    nblocks = N // B
    nsub = B // _SUB
    capacity = int(1.0 * ((N + E - 1) // E))

    idx, gate, pref, laux = pl.pallas_call(
        _router_body,
        grid=(nblocks,),
        in_specs=[
            pl.BlockSpec((B, D), lambda i: (i, 0)),
            pl.BlockSpec((E, D), lambda i: (0, 0)),
        ],
        out_specs=[
            pl.BlockSpec((B, 1), lambda i: (i, 0)),
            pl.BlockSpec((B, 1), lambda i: (i, 0)),
            pl.BlockSpec((1, nsub, E), lambda i: (i, 0, 0)),
            pl.BlockSpec((1, 1), lambda i: (0, 0)),
        ],
        out_shape=[
            jax.ShapeDtypeStruct((N, 1), jnp.int32),
            jax.ShapeDtypeStruct((N, 1), jnp.float32),
            jax.ShapeDtypeStruct((nblocks, nsub, E), jnp.int32),
            jax.ShapeDtypeStruct((1, 1), jnp.float32),
        ],
        scratch_shapes=[
            pltpu.VMEM((1, E), jnp.float32),
            pltpu.VMEM((1, E), jnp.float32),
        ],
    )(input, W)

    loc_kernel = _make_locations_kernel(N, E, 2, 16)
    loc = loc_kernel(idx[:, 0], pref.reshape(nblocks * nsub, E))

    return (laux[0, 0],
            jnp.asarray(capacity, dtype=jnp.int32),
            jnp.asarray(E, dtype=jnp.int32),
            idx[:, 0],
            loc,
            gate[:, 0])


# speedup vs baseline: 1.0676x; 1.0157x over previous
import functools

import jax
import jax.numpy as jnp
from jax import lax
from jax.experimental import pallas as pl
from jax.experimental.pallas import tpu as pltpu
from jax.experimental.pallas import tpu_sc as plsc

_SUB = 256
_LANES = 16


def _router_body(x_ref, w_ref, idx_ref, gate_ref, pref_ref, laux_ref,
                 cnt_ref, me_ref):
    i = pl.program_id(0)
    nblocks = pl.num_programs(0)
    B = x_ref.shape[0]
    E = w_ref.shape[0]

    @pl.when(i == 0)
    def _init():
        cnt_ref[...] = jnp.zeros_like(cnt_ref)
        me_ref[...] = jnp.zeros_like(me_ref)

    x = x_ref[...]
    w = w_ref[...]
    logits = jax.lax.dot_general(
        x, w, (((1,), (1,)), ((), ())),
        preferred_element_type=jnp.float32)

    rowmax = jnp.max(logits, axis=1, keepdims=True)
    e = jnp.exp(logits - rowmax)
    s = jnp.sum(e, axis=1, keepdims=True)
    gate_ref[...] = 1.0 / s

    lane = jax.lax.broadcasted_iota(jnp.int32, (B, E), 1)
    idx = jnp.min(jnp.where(logits == rowmax, lane, E), axis=1,
                  keepdims=True)
    idx_ref[...] = idx

    mask = (lane == idx).astype(jnp.float32)

    nsub = B // _SUB
    carry = cnt_ref[...]
    rows = []
    for k in range(nsub):
        rows.append(carry)
        carry = carry + jnp.sum(mask[k * _SUB:(k + 1) * _SUB], axis=0,
                                keepdims=True)
    pref_ref[...] = (jnp.concatenate(rows, axis=0)
                     .astype(jnp.int32).reshape(1, nsub, E))
    cnt_ref[...] = carry

    me_ref[...] = me_ref[...] + jnp.sum(e / s, axis=0, keepdims=True)

    @pl.when(i == nblocks - 1)
    def _fin():
        n_tok = B * nblocks
        ce = carry
        me = me_ref[...]
        laux_ref[...] = (jnp.sum(me * ce, keepdims=True)
                         * (E / (n_tok * n_tok))).reshape(1, 1)


def _make_locations_kernel(N, E, NC, NS):
    NW = NC * NS
    chunk = N // NW
    ngroups = chunk // _LANES
    mesh = plsc.VectorSubcoreMesh(core_axis_name="c", subcore_axis_name="s",
                                  num_cores=NC, num_subcores=NS)

    @functools.partial(
        pl.kernel, mesh=mesh,
        out_type=jax.ShapeDtypeStruct((N,), jnp.int32),
        compiler_params=pltpu.CompilerParams(needs_layout_passes=False),
        scratch_types=[
            pltpu.VMEM((chunk,), jnp.int32),
            pltpu.VMEM((E,), jnp.int32),
            pltpu.VMEM((chunk,), jnp.int32),
            pltpu.SemaphoreType.DMA,
            pltpu.SemaphoreType.DMA,
        ],
    )
    def _loc_kernel(idx_hbm, pref_hbm, loc_hbm, idx_v, cnt_v, loc_v,
                    sem1, sem2):
        w = lax.axis_index("s") * NC + lax.axis_index("c")
        base = w * chunk
        c1 = pltpu.async_copy(idx_hbm.at[pl.ds(base, chunk)], idx_v, sem1)
        c2 = pltpu.async_copy(pref_hbm.at[w], cnt_v, sem2)
        c1.wait()
        c2.wait()

        for g in range(ngroups):
            ids = idx_v[pl.ds(g * _LANES, _LANES)]
            cnt16, last = plsc.scan_count(ids)
            old = plsc.load_gather(cnt_v, [ids])
            loc_v[pl.ds(g * _LANES, _LANES)] = old + cnt16 - 1
            plsc.store_scatter(cnt_v, [ids], old + cnt16, mask=last)
        pltpu.sync_copy(loc_v, loc_hbm.at[pl.ds(base, chunk)])

    return _loc_kernel


def kernel(input, W):
    N, D = input.shape
    E = W.shape[0]
    B = 1024
    nblocks = N // B
    nsub = B // _SUB
    capacity = int(1.0 * ((N + E - 1) // E))

    idx, gate, pref, laux = pl.pallas_call(
        _router_body,
        grid=(nblocks,),
        in_specs=[
            pl.BlockSpec((B, D), lambda i: (i, 0)),
            pl.BlockSpec((E, D), lambda i: (0, 0)),
        ],
        out_specs=[
            pl.BlockSpec((B, 1), lambda i: (i, 0)),
            pl.BlockSpec((B, 1), lambda i: (i, 0)),
            pl.BlockSpec((1, nsub, E), lambda i: (i, 0, 0)),
            pl.BlockSpec((1, 1), lambda i: (0, 0)),
        ],
        out_shape=[
            jax.ShapeDtypeStruct((N, 1), jnp.int32),
            jax.ShapeDtypeStruct((N, 1), jnp.float32),
            jax.ShapeDtypeStruct((nblocks, nsub, E), jnp.int32),
            jax.ShapeDtypeStruct((1, 1), jnp.float32),
        ],
        scratch_shapes=[
            pltpu.VMEM((1, E), jnp.float32),
            pltpu.VMEM((1, E), jnp.float32),
        ],
    )(input, W)

    loc_kernel = _make_locations_kernel(N, E, 1, 16)
    loc = loc_kernel(idx[:, 0], pref.reshape(nblocks * nsub, E))

    return (laux[0, 0],
            jnp.asarray(capacity, dtype=jnp.int32),
            jnp.asarray(E, dtype=jnp.int32),
            idx[:, 0],
            loc,
            gate[:, 0])
